# Initial kernel scaffold; baseline (speedup 1.0000x reference)
import functools

import jax
import jax.numpy as jnp
from jax import lax
from jax.experimental import pallas as pl
from jax.experimental.pallas import tpu as pltpu
from jax.experimental.pallas import tpu_sc as plsc

VOCAB = 1000000
EMBED_DIM = 32
BATCH = 4096
SEQ = 200


def _embed_body(x_hbm, table_hbm, out_hbm, idx_v, rows_v, trans_v, sem):
    info = plsc.get_sparse_core_info()
    nc, ns = info.num_cores, info.num_subcores
    nw = nc * ns
    b_per_w = BATCH // nw

    wid = lax.axis_index("s") * nc + lax.axis_index("c")
    base = wid * b_per_w

    pltpu.sync_copy(x_hbm.at[pl.ds(base, b_per_w)], idx_v)

    lane = lax.iota(jnp.int32, 16)
    c_lo = lane
    c_hi = lane + 16

    def do_batch(b, carry):
        pltpu.async_copy(table_hbm.at[idx_v.at[b]], rows_v, sem).wait()

        def t_step(i, c2):
            for j in range(8):
                l = i * 8 + j
                l_vec = jnp.full((16,), l, jnp.int32)
                v0 = rows_v[l, pl.ds(0, 16)]
                v1 = rows_v[l, pl.ds(16, 16)]
                plsc.store_scatter(trans_v, [c_lo, l_vec], v0)
                plsc.store_scatter(trans_v, [c_hi, l_vec], v1)
            return c2

        lax.fori_loop(0, SEQ // 8, t_step, 0)

        pltpu.sync_copy(trans_v, out_hbm.at[base + b])
        return carry

    lax.fori_loop(0, b_per_w, do_batch, 0)


def kernel(x, table):
    info = plsc.get_sparse_core_info()
    nw = info.num_cores * info.num_subcores
    b_per_w = BATCH // nw
    mesh = plsc.VectorSubcoreMesh(core_axis_name="c", subcore_axis_name="s")

    f = functools.partial(
        pl.kernel,
        mesh=mesh,
        out_type=jax.ShapeDtypeStruct((BATCH, EMBED_DIM, SEQ), jnp.float32),
        scratch_types=[
            pltpu.VMEM((b_per_w, SEQ), jnp.int32),
            pltpu.VMEM((SEQ, EMBED_DIM), jnp.float32),
            pltpu.VMEM((EMBED_DIM, SEQ), jnp.float32),
            pltpu.SemaphoreType.DMA,
        ],
    )(_embed_body)
    return f(x, table)


# speedup vs baseline: 1.3913x; 1.3913x over previous
import functools

import jax
import jax.numpy as jnp
from jax import lax
from jax.experimental import pallas as pl
from jax.experimental.pallas import tpu as pltpu
from jax.experimental.pallas import tpu_sc as plsc

VOCAB = 1000000
EMBED_DIM = 32
BATCH = 4096
SEQ = 200


def _embed_body(x_hbm, table_hbm, out_hbm, idx_v, rows_v, trans_v, sem):
    info = plsc.get_sparse_core_info()
    nc, ns = info.num_cores, info.num_subcores
    nw = nc * ns
    b_per_w = BATCH // nw

    wid = lax.axis_index("s") * nc + lax.axis_index("c")
    base = wid * b_per_w

    pltpu.sync_copy(x_hbm.at[pl.ds(base, b_per_w)], idx_v)

    lane = lax.iota(jnp.int32, 16)
    c_lo = lane
    c_hi = lane + 16

    def do_batch(b, carry):
        pltpu.async_copy(table_hbm.at[idx_v.at[b]], rows_v, sem).wait()

        def t_step(i, c2):
            for j in range(8):
                l = i * 8 + j
                l_vec = jnp.full((16,), l, jnp.int32)
                v0 = rows_v[l, pl.ds(0, 16)]
                v1 = rows_v[l, pl.ds(16, 16)]
                plsc.store_scatter(trans_v, [c_lo, l_vec], v0)
                plsc.store_scatter(trans_v, [c_hi, l_vec], v1)
            return c2

        lax.fori_loop(0, SEQ // 8, t_step, 0)

        pltpu.sync_copy(trans_v, out_hbm.at[base + b])
        return carry

    lax.fori_loop(0, b_per_w, do_batch, 0)


def kernel(x, table):
    info = plsc.get_sparse_core_info()
    nw = info.num_cores * info.num_subcores
    b_per_w = BATCH // nw
    mesh = plsc.VectorSubcoreMesh(core_axis_name="c", subcore_axis_name="s")

    f = functools.partial(
        pl.kernel,
        mesh=mesh,
        compiler_params=pltpu.CompilerParams(
            use_tc_tiling_on_sc=False, needs_layout_passes=False),
        out_type=jax.ShapeDtypeStruct((BATCH, EMBED_DIM, SEQ), jnp.float32),
        scratch_types=[
            pltpu.VMEM((b_per_w, SEQ), jnp.int32),
            pltpu.VMEM((SEQ, EMBED_DIM), jnp.float32),
            pltpu.VMEM((EMBED_DIM, SEQ), jnp.float32),
            pltpu.SemaphoreType.DMA,
        ],
    )(_embed_body)
    return f(x, table)


# speedup vs baseline: 1.6050x; 1.1536x over previous
import functools

import jax
import jax.numpy as jnp
from jax import lax
from jax.experimental import pallas as pl
from jax.experimental.pallas import tpu as pltpu
from jax.experimental.pallas import tpu_sc as plsc

VOCAB = 1000000
EMBED_DIM = 32
BATCH = 4096
SEQ = 200

G = 2
UNROLL = 4
CH_IDX = G * SEQ
CH_OUT = G * EMBED_DIM * SEQ


def _embed_body(x_hbm, table_hbm, out_hbm, idx_v, rows0, rows1, trans0,
                trans1, gsem0, gsem1, osem0, osem1):
    info = plsc.get_sparse_core_info()
    nc, ns = info.num_cores, info.num_subcores
    nw = nc * ns
    b_per_w = BATCH // nw
    n_chunks = b_per_w // G

    wid = lax.axis_index("s") * nc + lax.axis_index("c")
    base = wid * b_per_w
    ibase = base * SEQ
    obase = base * EMBED_DIM * SEQ

    pltpu.sync_copy(x_hbm.at[pl.ds(ibase, b_per_w * SEQ)], idx_v)

    rows = (rows0, rows1)
    trans = (trans0, trans1)
    gsem = (gsem0, gsem1)
    osem = (osem0, osem1)

    lane = lax.iota(jnp.int32, 16)
    p_lo = lane * SEQ
    p_hi = (lane + 16) * SEQ

    def start_gather(c, k):
        pltpu.make_async_copy(
            table_hbm.at[idx_v.at[pl.ds(c * CH_IDX, CH_IDX)]],
            rows[k], gsem[k]).start()

    def wait_gather(k):
        pltpu.make_async_copy(
            table_hbm.at[idx_v.at[pl.ds(0, CH_IDX)]],
            rows[k], gsem[k]).wait()

    def start_out(c, k):
        pltpu.make_async_copy(
            trans[k], out_hbm.at[pl.ds(obase + c * CH_OUT, CH_OUT)],
            osem[k]).start()

    def wait_out(k):
        pltpu.make_async_copy(
            trans[k], out_hbm.at[pl.ds(obase, CH_OUT)], osem[k]).wait()

    def transpose(k):
        rv, tv = rows[k], trans[k]

        def t_step(i, c2):
            for dj in range(UNROLL):
                l = i * UNROLL + dj
                for g in range(G):
                    l2 = g * SEQ + l
                    off = g * EMBED_DIM * SEQ + l
                    v0 = rv[l2, pl.ds(0, 16)]
                    v1 = rv[l2, pl.ds(16, 16)]
                    plsc.store_scatter(tv, [p_lo + off], v0)
                    plsc.store_scatter(tv, [p_hi + off], v1)
            return c2

        lax.fori_loop(0, SEQ // UNROLL, t_step, 0)

    start_gather(0, 0)

    def pair(ci, carry):
        c0 = ci * 2

        start_gather(c0 + 1, 1)
        wait_gather(0)

        @pl.when(ci > 0)
        def _():
            wait_out(0)

        transpose(0)
        start_out(c0, 0)

        @pl.when(c0 + 2 < n_chunks)
        def _():
            start_gather(c0 + 2, 0)

        wait_gather(1)

        @pl.when(ci > 0)
        def _():
            wait_out(1)

        transpose(1)
        start_out(c0 + 1, 1)
        return carry

    lax.fori_loop(0, n_chunks // 2, pair, 0)
    wait_out(0)
    wait_out(1)


def kernel(x, table):
    info = plsc.get_sparse_core_info()
    nw = info.num_cores * info.num_subcores
    b_per_w = BATCH // nw
    mesh = plsc.VectorSubcoreMesh(core_axis_name="c", subcore_axis_name="s")

    f = functools.partial(
        pl.kernel,
        mesh=mesh,
        compiler_params=pltpu.CompilerParams(
            use_tc_tiling_on_sc=False, needs_layout_passes=False),
        out_type=jax.ShapeDtypeStruct((BATCH * EMBED_DIM * SEQ,),
                                      jnp.float32),
        scratch_types=[
            pltpu.VMEM((b_per_w * SEQ,), jnp.int32),
            pltpu.VMEM((CH_IDX, EMBED_DIM), jnp.float32),
            pltpu.VMEM((CH_IDX, EMBED_DIM), jnp.float32),
            pltpu.VMEM((CH_OUT,), jnp.float32),
            pltpu.VMEM((CH_OUT,), jnp.float32),
            pltpu.SemaphoreType.DMA,
            pltpu.SemaphoreType.DMA,
            pltpu.SemaphoreType.DMA,
            pltpu.SemaphoreType.DMA,
        ],
    )(_embed_body)
    out_flat = f(x.reshape(BATCH * SEQ), table)
    return out_flat.reshape(BATCH, EMBED_DIM, SEQ)
